# baseline (speedup 1.0000x reference)
import functools
import jax
import jax.numpy as jnp
from jax.experimental import pallas as pl
from jax.experimental.pallas import tpu as pltpu

N_NODES = 100000
N_EDGES = 400000
DIM = 128

NODE_BLK = 1000
EDGE_BLK = 2000


def _node_dense_body(xs_ref, xd_ref, mem_ref, ws_ref, wd_ref, wh_ref, bias_ref,
                     h_ref, p_ref):
    h = (mem_ref[...]
         + jnp.dot(xs_ref[...], ws_ref[...], preferred_element_type=jnp.float32)
         + jnp.dot(xd_ref[...], wd_ref[...], preferred_element_type=jnp.float32)
         + bias_ref[...])
    h_ref[...] = h
    p_ref[...] = jnp.dot(h, wh_ref[...], preferred_element_type=jnp.float32)


def _node_dense(xs, xd, mem, w_src, w_dst, w_h, bias):
    n = xs.shape[0]
    grid = (n // NODE_BLK,)
    blk = lambda i: (i, 0)
    full = lambda i: (0, 0)
    return pl.pallas_call(
        _node_dense_body,
        grid=grid,
        in_specs=[
            pl.BlockSpec((NODE_BLK, DIM), blk),
            pl.BlockSpec((NODE_BLK, DIM), blk),
            pl.BlockSpec((NODE_BLK, DIM), blk),
            pl.BlockSpec((DIM, DIM), full),
            pl.BlockSpec((DIM, DIM), full),
            pl.BlockSpec((DIM, DIM), full),
            pl.BlockSpec((1, DIM), full),
        ],
        out_specs=[
            pl.BlockSpec((NODE_BLK, DIM), blk),
            pl.BlockSpec((NODE_BLK, DIM), blk),
        ],
        out_shape=[
            jax.ShapeDtypeStruct((n, DIM), jnp.float32),
            jax.ShapeDtypeStruct((n, DIM), jnp.float32),
        ],
    )(xs, xd, mem, w_src, w_dst, w_h, bias)


def _edge_dense_body(psrc_ref, ef_ref, we_ref, b_ref, m_ref):
    q = jnp.dot(ef_ref[...], we_ref[...], preferred_element_type=jnp.float32)
    m_ref[...] = jnp.maximum(psrc_ref[...] + q + b_ref[...], 0.0)


def _edge_dense(p_src, ef, w_e, b_msg):
    e = p_src.shape[0]
    grid = (e // EDGE_BLK,)
    blk = lambda i: (i, 0)
    full = lambda i: (0, 0)
    fdim = ef.shape[1]
    return pl.pallas_call(
        _edge_dense_body,
        grid=grid,
        in_specs=[
            pl.BlockSpec((EDGE_BLK, DIM), blk),
            pl.BlockSpec((EDGE_BLK, fdim), blk),
            pl.BlockSpec((fdim, DIM), full),
            pl.BlockSpec((1, DIM), full),
        ],
        out_specs=pl.BlockSpec((EDGE_BLK, DIM), blk),
        out_shape=jax.ShapeDtypeStruct((e, DIM), jnp.float32),
    )(p_src, ef, w_e, b_msg)


def _upd_dense_body(h_ref, agg_ref, w1_ref, w2_ref, b_ref, out_ref):
    acc = (jnp.dot(h_ref[...], w1_ref[...], preferred_element_type=jnp.float32)
           + jnp.dot(agg_ref[...], w2_ref[...], preferred_element_type=jnp.float32)
           + b_ref[...])
    out_ref[...] = jnp.maximum(acc, 0.0)


def _upd_dense(h, agg, w1, w2, bias):
    n = h.shape[0]
    grid = (n // NODE_BLK,)
    blk = lambda i: (i, 0)
    full = lambda i: (0, 0)
    return pl.pallas_call(
        _upd_dense_body,
        grid=grid,
        in_specs=[
            pl.BlockSpec((NODE_BLK, DIM), blk),
            pl.BlockSpec((NODE_BLK, DIM), blk),
            pl.BlockSpec((DIM, DIM), full),
            pl.BlockSpec((DIM, DIM), full),
            pl.BlockSpec((1, DIM), full),
        ],
        out_specs=pl.BlockSpec((NODE_BLK, DIM), blk),
        out_shape=jax.ShapeDtypeStruct((n, DIM), jnp.float32),
    )(h, agg, w1, w2, bias)


def kernel(edge_index, t, msg, x_src, x_dst, node_memory, last_update,
           W_src, b_src, W_dst, b_dst, time_w, time_b, W_msg, b_msg,
           W_upd, b_upd):
    src, dst = edge_index[0], edge_index[1]

    cache = jnp.zeros((N_NODES, DIM), dtype=x_src.dtype)
    cache = cache.at[src].set(x_src)
    cache = cache.at[dst].set(x_dst)
    src_mask = jnp.zeros((N_NODES,), dtype=bool).at[src].set(True)
    dst_mask = jnp.zeros((N_NODES,), dtype=bool).at[dst].set(True)
    xs = jnp.where(src_mask[:, None], cache, 0.0)
    xd = jnp.where(dst_mask[:, None], cache, 0.0)

    w_h = W_msg[:DIM]
    w_e = W_msg[DIM:]
    bias_h = (b_src + b_dst).reshape(1, DIM)
    h, p = _node_dense(xs, xd, node_memory, W_src, W_dst, w_h, bias_h)

    rel_t = last_update[src] - t.astype(jnp.float32)
    rel_enc = jnp.cos(rel_t[:, None] * time_w[None, :] + time_b[None, :])
    ef = jnp.concatenate([msg, rel_enc], axis=-1)

    p_src = p[src]
    m = _edge_dense(p_src, ef, w_e, b_msg.reshape(1, DIM))

    agg = jax.ops.segment_sum(m, dst, num_segments=N_NODES)

    h_out = _upd_dense(h, agg, W_upd[:DIM], W_upd[DIM:], b_upd.reshape(1, DIM))
    return h_out[src], h_out[dst]


# speedup vs baseline: 1.2768x; 1.2768x over previous
import functools
import jax
import jax.numpy as jnp
from jax import lax
from jax.experimental import pallas as pl
from jax.experimental.pallas import tpu as pltpu
from jax.experimental.pallas import tpu_sc as plsc

N_NODES = 100000
N_EDGES = 400000
DIM = 128

NODE_BLK = 1000
EDGE_BLK = 2000

SC_WORKERS = 25
SC_PER_W = 16000
SC_CHUNK = 320

_SC_MESH = plsc.VectorSubcoreMesh(core_axis_name="c", subcore_axis_name="s")


def _row_gather(table, idx):
    e = idx.shape[0]
    d = table.shape[1]

    @functools.partial(
        pl.kernel, mesh=_SC_MESH,
        out_type=jax.ShapeDtypeStruct((e, d), jnp.float32),
        scratch_types=[
            pltpu.VMEM((SC_CHUNK,), jnp.int32),
            pltpu.VMEM((SC_CHUNK, d), jnp.float32),
            pltpu.SemaphoreType.DMA,
        ],
    )
    def k(table_hbm, idx_hbm, out_hbm, idx_v, rows_v, sem):
        wid = lax.axis_index("s") * 2 + lax.axis_index("c")

        @pl.when(wid < SC_WORKERS)
        def _():
            base = wid * SC_PER_W

            def body(j, carry):
                off = base + j * SC_CHUNK
                pltpu.sync_copy(idx_hbm.at[pl.ds(off, SC_CHUNK)], idx_v)
                pltpu.async_copy(table_hbm.at[idx_v], rows_v, sem).wait()
                pltpu.sync_copy(rows_v, out_hbm.at[pl.ds(off, SC_CHUNK)])
                return carry

            lax.fori_loop(0, SC_PER_W // SC_CHUNK, body, 0)

    return k(table, idx)


def _elem_gather(table, idx):
    e = idx.shape[0]

    @functools.partial(
        pl.kernel, mesh=_SC_MESH,
        out_type=jax.ShapeDtypeStruct((e,), jnp.float32),
        scratch_types=[
            pltpu.VMEM((SC_CHUNK,), jnp.int32),
            pltpu.VMEM((SC_CHUNK,), jnp.float32),
            pltpu.SemaphoreType.DMA,
        ],
    )
    def k(table_hbm, idx_hbm, out_hbm, idx_v, vals_v, sem):
        wid = lax.axis_index("s") * 2 + lax.axis_index("c")

        @pl.when(wid < SC_WORKERS)
        def _():
            base = wid * SC_PER_W

            def body(j, carry):
                off = base + j * SC_CHUNK
                pltpu.sync_copy(idx_hbm.at[pl.ds(off, SC_CHUNK)], idx_v)
                pltpu.async_copy(table_hbm.at[idx_v], vals_v, sem).wait()
                pltpu.sync_copy(vals_v, out_hbm.at[pl.ds(off, SC_CHUNK)])
                return carry

            lax.fori_loop(0, SC_PER_W // SC_CHUNK, body, 0)

    return k(table, idx)


def _node_dense_body(xs_ref, xd_ref, mem_ref, ws_ref, wd_ref, wh_ref, bias_ref,
                     h_ref, p_ref):
    h = (mem_ref[...]
         + jnp.dot(xs_ref[...], ws_ref[...], preferred_element_type=jnp.float32)
         + jnp.dot(xd_ref[...], wd_ref[...], preferred_element_type=jnp.float32)
         + bias_ref[...])
    h_ref[...] = h
    p_ref[...] = jnp.dot(h, wh_ref[...], preferred_element_type=jnp.float32)


def _node_dense(xs, xd, mem, w_src, w_dst, w_h, bias):
    n = xs.shape[0]
    grid = (n // NODE_BLK,)
    blk = lambda i: (i, 0)
    full = lambda i: (0, 0)
    return pl.pallas_call(
        _node_dense_body,
        grid=grid,
        in_specs=[
            pl.BlockSpec((NODE_BLK, DIM), blk),
            pl.BlockSpec((NODE_BLK, DIM), blk),
            pl.BlockSpec((NODE_BLK, DIM), blk),
            pl.BlockSpec((DIM, DIM), full),
            pl.BlockSpec((DIM, DIM), full),
            pl.BlockSpec((DIM, DIM), full),
            pl.BlockSpec((1, DIM), full),
        ],
        out_specs=[
            pl.BlockSpec((NODE_BLK, DIM), blk),
            pl.BlockSpec((NODE_BLK, DIM), blk),
        ],
        out_shape=[
            jax.ShapeDtypeStruct((n, DIM), jnp.float32),
            jax.ShapeDtypeStruct((n, DIM), jnp.float32),
        ],
    )(xs, xd, mem, w_src, w_dst, w_h, bias)


def _edge_msg_body(psrc_ref, msg_ref, lu_ref, t_ref, tw_ref, tb_ref,
                   we1_ref, we2_ref, b_ref, m_ref):
    rel = lu_ref[...] - t_ref[...].astype(jnp.float32)
    enc = jnp.cos(rel * tw_ref[...] + tb_ref[...])
    q = (jnp.dot(msg_ref[...], we1_ref[...], preferred_element_type=jnp.float32)
         + jnp.dot(enc, we2_ref[...], preferred_element_type=jnp.float32))
    m_ref[...] = jnp.maximum(psrc_ref[...] + q + b_ref[...], 0.0)


def _edge_msg(p_src, msg, lu_src, t, time_w, time_b, w_e1, w_e2, b_msg):
    e = p_src.shape[0]
    ed = msg.shape[1]
    td = time_w.shape[1]
    grid = (e // EDGE_BLK,)
    blk = lambda i: (i, 0)
    full = lambda i: (0, 0)
    return pl.pallas_call(
        _edge_msg_body,
        grid=grid,
        in_specs=[
            pl.BlockSpec((EDGE_BLK, DIM), blk),
            pl.BlockSpec((EDGE_BLK, ed), blk),
            pl.BlockSpec((EDGE_BLK, 1), blk),
            pl.BlockSpec((EDGE_BLK, 1), blk),
            pl.BlockSpec((1, td), full),
            pl.BlockSpec((1, td), full),
            pl.BlockSpec((ed, DIM), full),
            pl.BlockSpec((td, DIM), full),
            pl.BlockSpec((1, DIM), full),
        ],
        out_specs=pl.BlockSpec((EDGE_BLK, DIM), blk),
        out_shape=jax.ShapeDtypeStruct((e, DIM), jnp.float32),
    )(p_src, msg, lu_src, t, time_w, time_b, w_e1, w_e2, b_msg)


def _upd_dense_body(h_ref, agg_ref, w1_ref, w2_ref, b_ref, out_ref):
    acc = (jnp.dot(h_ref[...], w1_ref[...], preferred_element_type=jnp.float32)
           + jnp.dot(agg_ref[...], w2_ref[...], preferred_element_type=jnp.float32)
           + b_ref[...])
    out_ref[...] = jnp.maximum(acc, 0.0)


def _upd_dense(h, agg, w1, w2, bias):
    n = h.shape[0]
    grid = (n // NODE_BLK,)
    blk = lambda i: (i, 0)
    full = lambda i: (0, 0)
    return pl.pallas_call(
        _upd_dense_body,
        grid=grid,
        in_specs=[
            pl.BlockSpec((NODE_BLK, DIM), blk),
            pl.BlockSpec((NODE_BLK, DIM), blk),
            pl.BlockSpec((DIM, DIM), full),
            pl.BlockSpec((DIM, DIM), full),
            pl.BlockSpec((1, DIM), full),
        ],
        out_specs=pl.BlockSpec((NODE_BLK, DIM), blk),
        out_shape=jax.ShapeDtypeStruct((n, DIM), jnp.float32),
    )(h, agg, w1, w2, bias)


def kernel(edge_index, t, msg, x_src, x_dst, node_memory, last_update,
           W_src, b_src, W_dst, b_dst, time_w, time_b, W_msg, b_msg,
           W_upd, b_upd):
    src, dst = edge_index[0], edge_index[1]

    cache = jnp.zeros((N_NODES, DIM), dtype=x_src.dtype)
    cache = cache.at[src].set(x_src)
    cache = cache.at[dst].set(x_dst)
    src_mask = jnp.zeros((N_NODES,), dtype=bool).at[src].set(True)
    dst_mask = jnp.zeros((N_NODES,), dtype=bool).at[dst].set(True)
    xs = jnp.where(src_mask[:, None], cache, 0.0)
    xd = jnp.where(dst_mask[:, None], cache, 0.0)

    w_h = W_msg[:DIM]
    bias_h = (b_src + b_dst).reshape(1, DIM)
    h, p = _node_dense(xs, xd, node_memory, W_src, W_dst, w_h, bias_h)

    lu_src = _elem_gather(last_update, src)
    p_src = _row_gather(p, src)
    m = _edge_msg(p_src, msg, lu_src.reshape(-1, 1), t.reshape(-1, 1),
                  time_w.reshape(1, -1), time_b.reshape(1, -1),
                  W_msg[DIM:DIM + 16], W_msg[DIM + 16:],
                  b_msg.reshape(1, DIM))

    agg = jax.ops.segment_sum(m, dst, num_segments=N_NODES)

    h_out = _upd_dense(h, agg, W_upd[:DIM], W_upd[DIM:], b_upd.reshape(1, DIM))

    h_src = _row_gather(h_out, src)
    h_dst = _row_gather(h_out, dst)
    return h_src, h_dst
